# baseline (device time: 11867 ns/iter reference)
import jax
import jax.numpy as jnp
from jax import lax
from jax.experimental import pallas as pl
from jax.experimental.pallas import tpu as pltpu

N_DEV = 16


def kernel(x):
    m_rows, n_cols = x.shape

    def body(x_ref, out_ref, gather_ref, send_sems, recv_sems):
        my = lax.axis_index("i")

        barrier_sem = pltpu.get_barrier_semaphore()
        for off in range(1, N_DEV):
            peer = lax.rem(my + off, N_DEV)
            pl.semaphore_signal(
                barrier_sem, inc=1,
                device_id=(peer,), device_id_type=pl.DeviceIdType.MESH,
            )

        xv = x_ref[...].astype(jnp.float32)
        m_loc = jnp.max(xv, axis=1, keepdims=True)
        e = jnp.exp(xv - m_loc)
        s_loc = jnp.sum(e, axis=1, keepdims=True)
        pad = jnp.zeros((m_rows, 6), jnp.float32)
        stats_row = jnp.transpose(
            jnp.concatenate([m_loc, s_loc, pad], axis=1), (1, 0)
        )
        gather_ref[my] = stats_row

        pl.semaphore_wait(barrier_sem, N_DEV - 1)

        sends = []
        for off in range(1, N_DEV):
            peer = lax.rem(my + off, N_DEV)
            rdma = pltpu.make_async_remote_copy(
                src_ref=gather_ref.at[my],
                dst_ref=gather_ref.at[my],
                send_sem=send_sems.at[peer],
                recv_sem=recv_sems.at[my],
                device_id=(peer,),
                device_id_type=pl.DeviceIdType.MESH,
            )
            rdma.start()
            sends.append(rdma)
        for off in range(1, N_DEV):
            peer = lax.rem(my + off, N_DEV)
            recv = pltpu.make_async_remote_copy(
                src_ref=gather_ref.at[peer],
                dst_ref=gather_ref.at[peer],
                send_sem=send_sems.at[peer],
                recv_sem=recv_sems.at[peer],
                device_id=(peer,),
                device_id_type=pl.DeviceIdType.MESH,
            )
            recv.wait_recv()

        g = gather_ref[...]
        m_all = g[:, 0:1, :]
        s_all = g[:, 1:2, :]
        m_glob = jnp.max(m_all, axis=0)
        s_glob = jnp.sum(s_all * jnp.exp(m_all - m_glob[None]), axis=0)
        pad_row = jnp.zeros((6, m_rows), jnp.float32)
        res_col = jnp.transpose(
            jnp.concatenate([m_glob, s_glob, pad_row], axis=0), (1, 0)
        )
        corr = jnp.exp(m_loc - res_col[:, 0:1]) / res_col[:, 1:2]
        out_ref[...] = (e * corr).astype(out_ref.dtype)

        for rdma in sends:
            rdma.wait_send()

    return pl.pallas_call(
        body,
        out_shape=jax.ShapeDtypeStruct((m_rows, n_cols), jnp.float32),
        in_specs=[pl.BlockSpec(memory_space=pltpu.VMEM)],
        out_specs=pl.BlockSpec(memory_space=pltpu.VMEM),
        scratch_shapes=[
            pltpu.VMEM((N_DEV, 8, m_rows), jnp.float32),
            pltpu.SemaphoreType.DMA((N_DEV,)),
            pltpu.SemaphoreType.DMA((N_DEV,)),
        ],
        compiler_params=pltpu.CompilerParams(collective_id=0),
    )(x)


# device time: 10584 ns/iter; 1.1212x vs baseline; 1.1212x over previous
import jax
import jax.numpy as jnp
from jax import lax
from jax.experimental import pallas as pl
from jax.experimental.pallas import tpu as pltpu

N_DEV = 16
N_GRP = 4


def kernel(x):
    m_rows, n_cols = x.shape
    g_rows = m_rows // N_GRP

    def body(x_ref, out_ref, gather_ref, send_sems, recv_sems):
        my = lax.axis_index("i")

        barrier_sem = pltpu.get_barrier_semaphore()
        for off in range(1, N_DEV):
            peer = lax.rem(my + off, N_DEV)
            pl.semaphore_signal(
                barrier_sem, inc=1,
                device_id=(peer,), device_id_type=pl.DeviceIdType.MESH,
            )

        xv = x_ref[...].astype(jnp.float32)
        cols = []
        for grp in range(N_GRP):
            xg = xv[grp * g_rows:(grp + 1) * g_rows, :]
            mg = jnp.max(xg, axis=1, keepdims=True)
            sg = jnp.sum(jnp.exp(xg - mg), axis=1, keepdims=True)
            cols.append((mg, sg))
        stats_col = jnp.concatenate(
            [mg for mg, _ in cols] + [sg for _, sg in cols], axis=1
        )
        gather_ref[my] = jnp.transpose(stats_col, (1, 0))

        pl.semaphore_wait(barrier_sem, N_DEV - 1)

        sends = []
        for off in range(1, N_DEV):
            peer = lax.rem(my + off, N_DEV)
            rdma = pltpu.make_async_remote_copy(
                src_ref=gather_ref.at[my],
                dst_ref=gather_ref.at[my],
                send_sem=send_sems.at[peer],
                recv_sem=recv_sems.at[my],
                device_id=(peer,),
                device_id_type=pl.DeviceIdType.MESH,
            )
            rdma.start()
            sends.append(rdma)

        e_grps = [
            jnp.exp(
                xv[grp * g_rows:(grp + 1) * g_rows, :] - cols[grp][0]
            )
            for grp in range(N_GRP)
        ]

        for off in range(1, N_DEV):
            peer = lax.rem(my + off, N_DEV)
            recv = pltpu.make_async_remote_copy(
                src_ref=gather_ref.at[peer],
                dst_ref=gather_ref.at[peer],
                send_sem=send_sems.at[peer],
                recv_sem=recv_sems.at[peer],
                device_id=(peer,),
                device_id_type=pl.DeviceIdType.MESH,
            )
            recv.wait_recv()

        g = gather_ref[...]
        m_all = g[:, :N_GRP, :]
        s_all = g[:, N_GRP:, :]
        m_glob = jnp.max(m_all, axis=0)
        s_glob = jnp.sum(s_all * jnp.exp(m_all - m_glob[None]), axis=0)
        res_col = jnp.transpose(
            jnp.concatenate([m_glob, s_glob], axis=0), (1, 0)
        )
        for grp in range(N_GRP):
            mg = res_col[:, grp:grp + 1]
            sg = res_col[:, N_GRP + grp:N_GRP + grp + 1]
            corr = jnp.exp(cols[grp][0] - mg) / sg
            out_ref[grp * g_rows:(grp + 1) * g_rows, :] = (
                e_grps[grp] * corr
            ).astype(out_ref.dtype)

        for rdma in sends:
            rdma.wait_send()

    return pl.pallas_call(
        body,
        out_shape=jax.ShapeDtypeStruct((m_rows, n_cols), jnp.float32),
        in_specs=[pl.BlockSpec(memory_space=pltpu.VMEM)],
        out_specs=pl.BlockSpec(memory_space=pltpu.VMEM),
        scratch_shapes=[
            pltpu.VMEM((N_DEV, 8, g_rows), jnp.float32),
            pltpu.SemaphoreType.DMA((N_DEV,)),
            pltpu.SemaphoreType.DMA((N_DEV,)),
        ],
        compiler_params=pltpu.CompilerParams(collective_id=0),
    )(x)
